# Initial kernel scaffold; baseline (speedup 1.0000x reference)
import jax
import jax.numpy as jnp
from jax.experimental import pallas as pl


def kernel(x, hierarchy_mapping, num_fine_nodes, W, b, gamma, beta):
    raise NotImplementedError("write your pallas kernel here")


# speedup vs baseline: 1.8693x; 1.8693x over previous
import functools

import jax
import jax.numpy as jnp
from jax import lax
from jax.experimental import pallas as pl
from jax.experimental.pallas import tpu as pltpu
from jax.experimental.pallas import tpu_sc as plsc


def _transform_body(x_ref, w_ref, b_ref, g_ref, bt_ref, o_ref):
    h = jnp.dot(x_ref[...], w_ref[...], preferred_element_type=jnp.float32)
    h = h + b_ref[...]
    m = jnp.mean(h, axis=-1, keepdims=True)
    v = jnp.mean(jnp.square(h - m), axis=-1, keepdims=True)
    h = (h - m) * lax.rsqrt(v + 1e-5) * g_ref[...] + bt_ref[...]
    o_ref[...] = h * jax.nn.sigmoid(h)


def _transform(xf, W, b, gamma, beta, blk):
    R, H = xf.shape
    assert R % blk == 0
    return pl.pallas_call(
        _transform_body,
        grid=(R // blk,),
        in_specs=[
            pl.BlockSpec((blk, H), lambda i: (i, 0)),
            pl.BlockSpec((H, H), lambda i: (0, 0)),
            pl.BlockSpec((1, H), lambda i: (0, 0)),
            pl.BlockSpec((1, H), lambda i: (0, 0)),
            pl.BlockSpec((1, H), lambda i: (0, 0)),
        ],
        out_specs=pl.BlockSpec((blk, H), lambda i: (i, 0)),
        out_shape=jax.ShapeDtypeStruct((R, H), jnp.float32),
    )(xf, W, b.reshape(1, H), gamma.reshape(1, H), beta.reshape(1, H))


_CHUNK = 128
_NW = 32


@functools.lru_cache(maxsize=None)
def _make_gather(T, D, rows, n_chunks_padded):
    cpw = n_chunks_padded // _NW
    full_chunks = rows // _CHUNK
    rem = rows % _CHUNK

    mesh = plsc.VectorSubcoreMesh(core_axis_name="c", subcore_axis_name="s")

    @functools.partial(
        pl.kernel,
        mesh=mesh,
        out_type=jax.ShapeDtypeStruct((rows, D), jnp.float32),
        scratch_types=[
            pltpu.VMEM((_CHUNK,), jnp.int32),
            pltpu.VMEM((_CHUNK, D), jnp.float32),
            pltpu.SemaphoreType.DMA,
        ],
    )
    def gather(table_hbm, idx_hbm, out_hbm, idx_v, rows_v, sem):
        wid = lax.axis_index("s") * 2 + lax.axis_index("c")

        def body(k, carry):
            chunk = wid * cpw + k

            @pl.when(chunk < full_chunks + (1 if rem else 0))
            def _():
                pltpu.sync_copy(idx_hbm.at[chunk], idx_v)
                pltpu.async_copy(table_hbm.at[idx_v], rows_v, sem).wait()

            @pl.when(chunk < full_chunks)
            def _():
                pltpu.sync_copy(rows_v, out_hbm.at[pl.ds(chunk * _CHUNK, _CHUNK)])

            if rem:
                @pl.when(chunk == full_chunks)
                def _():
                    pltpu.sync_copy(
                        rows_v.at[pl.ds(0, rem)],
                        out_hbm.at[pl.ds(full_chunks * _CHUNK, rem)],
                    )

            return carry

        lax.fori_loop(0, cpw, body, 0)

    return gather


def kernel(x, hierarchy_mapping, num_fine_nodes, W, b, gamma, beta):
    B, NC, Fm, H = x.shape
    NF = hierarchy_mapping.shape[0]
    D = Fm * H
    rows = B * NF

    xtf = _transform(x.reshape(-1, H), W, b, gamma, beta, blk=2000)
    table = xtf.reshape(B * NC, D)

    idx = (hierarchy_mapping[None, :].astype(jnp.int32)
           + NC * jnp.arange(B, dtype=jnp.int32)[:, None]).reshape(-1)
    n_chunks = -(-rows // _CHUNK)
    n_chunks_padded = -(-n_chunks // _NW) * _NW
    pad = n_chunks_padded * _CHUNK - rows
    if pad:
        idx = jnp.concatenate([idx, jnp.zeros((pad,), jnp.int32)])
    idx2 = idx.reshape(n_chunks_padded, _CHUNK)

    out = _make_gather(B * NC, D, rows, n_chunks_padded)(table, idx2)
    return out.reshape(B, NF, Fm, H)
